# baseline (speedup 1.0000x reference)
import functools

import jax
import jax.numpy as jnp
from jax import lax
from jax.experimental import pallas as pl
from jax.experimental.pallas import tpu as pltpu
from jax.experimental.pallas import tpu_sc as plsc

N = 50000
E = 800000
NUM_TYPES = 28
H = 64
OUT = 32
NC, NS, L = 2, 16, 16
NP = 50176
CH = 128
NROWS = 6400
EP = NROWS * CH
CH2 = 256
NROWS2 = EP // CH2
STRIPE = NP // NS
R = 512
G = NP // R


def _mesh():
    return plsc.VectorSubcoreMesh(core_axis_name="c", subcore_axis_name="s",
                                  num_cores=NC, num_subcores=NS)


def _zero_rows(buf, nrows, width):
    z = jnp.zeros((16,), jnp.float32)

    def body(i, _):
        for k in range(width // 16):
            buf[i, pl.ds(k * 16, 16)] = z
        return 0

    lax.fori_loop(0, nrows, body, 0)


def _zero_stripe(acc, base, zbuf):
    nz = zbuf.shape[0]
    nfull = STRIPE // nz
    rem = STRIPE - nfull * nz
    for k in range(nfull):
        pltpu.sync_copy(zbuf, acc.at[pl.ds(base + k * nz, nz)])
    if rem:
        pltpu.sync_copy(zbuf.at[pl.ds(0, rem)], acc.at[pl.ds(base + nfull * nz, rem)])


def _scatter_loop(tab, edges, r0, ib, rows, isems, gsems, ssems, acc, nchunks):
    def start_idx(j, k):
        pltpu.async_copy(edges.at[r0 + j], ib.at[k % 8], isems[k % 4])

    def wait_idx(j, k):
        pltpu.make_async_copy(edges.at[r0 + j], ib.at[k % 8], isems[k % 4]).wait()

    def start_gather(j, k):
        pltpu.async_copy(tab.at[ib.at[k % 8, 0]], rows[k % 8], gsems[k % 4])

    def wait_gather(j, k):
        pltpu.make_async_copy(tab.at[ib.at[k % 8, 0]], rows[k % 8],
                              gsems[k % 4]).wait()

    def start_scatter(j, k):
        pltpu.async_copy(rows[k % 8], acc.at[ib.at[k % 8, 1]], ssems[k % 4],
                         add=True)

    def wait_scatter(j, k):
        pltpu.make_async_copy(rows[k % 8], acc.at[ib.at[k % 8, 1]],
                              ssems[k % 4]).wait()

    def slot(j, k, first):
        wait_gather(j, k)
        start_scatter(j, k)
        if not first or k >= 3:
            wait_scatter(j - 3, k - 3)
        if not first or k >= 3:
            start_idx(j + 5, k + 5)
        wait_idx(j + 3, k + 3)
        start_gather(j + 3, k + 3)

    for j in range(8):
        start_idx(j, j)
    for j in range(3):
        wait_idx(j, j)
        start_gather(j, j)
    for k in range(8):
        slot(k, k, True)

    def it(i, _):
        for k in range(8):
            slot(8 * i + k, k, False)
        return 0

    nblk = nchunks // 8
    lax.fori_loop(1, nblk - 1, it, 0)
    for k in range(8):
        j = nchunks - 8 + k
        wait_gather(j, k)
        start_scatter(j, k)
        wait_scatter(j - 3, k - 3)
        if k < 3:
            start_idx(j + 5, k + 5)
        if k < 5:
            wait_idx(j + 3, k + 3)
            start_gather(j + 3, k + 3)
    wait_scatter(nchunks - 3, nchunks - 3)
    wait_scatter(nchunks - 2, nchunks - 2)
    wait_scatter(nchunks - 1, nchunks - 1)


def _scatter_loop2(tab, edges, r0, ib, rows, isems, gsems, ssems, acc, nchunks):
    def start_idx(j, k):
        pltpu.async_copy(edges.at[r0 + j], ib.at[k % 4], isems[k % 2])

    def wait_idx(j, k):
        pltpu.make_async_copy(edges.at[r0 + j], ib.at[k % 4], isems[k % 2]).wait()

    def start_gather(j, k):
        pltpu.async_copy(tab.at[ib.at[k % 4, 0]], rows[k % 2], gsems[k % 2])

    def wait_gather(j, k):
        pltpu.make_async_copy(tab.at[ib.at[k % 4, 0]], rows[k % 2],
                              gsems[k % 2]).wait()

    def start_scatter(j, k):
        pltpu.async_copy(rows[k % 2], acc.at[ib.at[k % 4, 1]], ssems[k % 2],
                         add=True)

    def wait_scatter(j, k):
        pltpu.make_async_copy(rows[k % 2], acc.at[ib.at[k % 4, 1]],
                              ssems[k % 2]).wait()

    def slot(j, k, first):
        wait_gather(j, k)
        start_scatter(j, k)
        if not first or k >= 1:
            wait_scatter(j - 1, k - 1)
        start_idx(j + 2, k + 2)
        wait_idx(j + 1, k + 1)
        start_gather(j + 1, k + 1)

    start_idx(0, 0)
    start_idx(1, 1)
    wait_idx(0, 0)
    start_gather(0, 0)
    for k in range(4):
        slot(k, k, True)

    def it(i, _):
        for k in range(4):
            slot(4 * i + k, k, False)
        return 0

    nblk = nchunks // 4
    lax.fori_loop(1, nblk - 1, it, 0)
    for k in range(4):
        j = nchunks - 4 + k
        wait_gather(j, k)
        start_scatter(j, k)
        wait_scatter(j - 1, k - 1)
        if k < 2:
            start_idx(j + 2, k + 2)
        if k < 3:
            wait_idx(j + 1, k + 1)
            start_gather(j + 1, k + 1)
    wait_scatter(nchunks - 1, nchunks - 1)


def _deg_kernel(dst2d):
    cpt = NROWS // (NC * NS)

    def body(dst_hbm, d0_hbm, d1_hbm, didx, ones, zbuf, acc):
        c = lax.axis_index("c")
        s = lax.axis_index("s")
        r0 = (c * NS + s) * cpt
        pltpu.sync_copy(dst_hbm.at[pl.ds(r0, cpt)], didx)
        _zero_rows(zbuf, CH, 16)
        base = s * STRIPE
        _zero_stripe(acc, base, zbuf)
        _zero_rows(ones, CH, 16)

        def setones(i, _):
            o = jnp.full((16,), 1.0, jnp.float32)
            ones[i, pl.ds(0, 16)] = o
            return 0

        lax.fori_loop(0, CH, setones, 0)
        plsc.subcore_barrier()

        def it(j, _):
            pltpu.sync_copy(ones, acc.at[didx.at[j]], add=True)
            return 0

        lax.fori_loop(0, cpt, it, 0)
        plsc.subcore_barrier()

        @pl.when(c == 0)
        def _():
            pltpu.sync_copy(acc.at[pl.ds(base, STRIPE)], d0_hbm.at[pl.ds(base, STRIPE)])

        @pl.when(c == 1)
        def _():
            pltpu.sync_copy(acc.at[pl.ds(base, STRIPE)], d1_hbm.at[pl.ds(base, STRIPE)])

    f = pl.kernel(
        body,
        out_type=[jax.ShapeDtypeStruct((NP, 16), jnp.float32),
                  jax.ShapeDtypeStruct((NP, 16), jnp.float32)],
        mesh=_mesh(),
        compiler_params=pltpu.CompilerParams(use_tc_tiling_on_sc=False),
        scratch_types=[
            pltpu.VMEM((cpt, CH), jnp.int32),
            pltpu.VMEM((CH, 16), jnp.float32),
            pltpu.VMEM((CH, 16), jnp.float32),
            pltpu.VMEM_SHARED((NP, 16), jnp.float32),
        ],
    )
    return f(dst2d)


_PROP_SCRATCH = [
    pltpu.VMEM((4, 2, CH2), jnp.int32),
    pltpu.VMEM((CH2, 32), jnp.float32),
    pltpu.VMEM((CH2, 32), jnp.float32),
    pltpu.VMEM_SHARED((NP, 32), jnp.float32),
] + [pltpu.SemaphoreType.DMA] * 6


def _prop_feat_split(edges3d, zs_lo, zs_hi):
    cpt = NROWS2 // NS

    def body(edges_hbm, lo_hbm, hi_hbm, ulo_hbm, uhi_hbm,
             ib, rows0, rows1, acc, *sems):
        c = lax.axis_index("c")
        s = lax.axis_index("s")
        r0 = s * cpt
        base = s * STRIPE
        @pl.when(c == 0)
        def _():
            pltpu.sync_copy(lo_hbm.at[pl.ds(base, STRIPE)], acc.at[pl.ds(base, STRIPE)])

        @pl.when(c == 1)
        def _():
            pltpu.sync_copy(hi_hbm.at[pl.ds(base, STRIPE)], acc.at[pl.ds(base, STRIPE)])

        plsc.subcore_barrier()
        rows = (rows0, rows1)
        isems, gsems, ssems = sems[0:2], sems[2:4], sems[4:6]

        @pl.when(c == 0)
        def _():
            _scatter_loop2(lo_hbm, edges_hbm, r0, ib, rows, isems, gsems,
                           ssems, acc, cpt)

        @pl.when(c == 1)
        def _():
            _scatter_loop2(hi_hbm, edges_hbm, r0, ib, rows, isems, gsems,
                           ssems, acc, cpt)

        plsc.subcore_barrier()

        @pl.when(c == 0)
        def _():
            pltpu.sync_copy(acc.at[pl.ds(base, STRIPE)], ulo_hbm.at[pl.ds(base, STRIPE)])

        @pl.when(c == 1)
        def _():
            pltpu.sync_copy(acc.at[pl.ds(base, STRIPE)], uhi_hbm.at[pl.ds(base, STRIPE)])

    f = pl.kernel(
        body,
        out_type=[jax.ShapeDtypeStruct((NP, 32), jnp.float32),
                  jax.ShapeDtypeStruct((NP, 32), jnp.float32)],
        mesh=_mesh(),
        compiler_params=pltpu.CompilerParams(use_tc_tiling_on_sc=False),
        scratch_types=_PROP_SCRATCH,
    )
    return f(edges3d, zs_lo, zs_hi)


def _prop_edge_split(edges3d, zs):
    cpt = NROWS2 // (NC * NS)

    def body(edges_hbm, zs_hbm, u0_hbm, u1_hbm,
             ib, rows0, rows1, acc, *sems):
        c = lax.axis_index("c")
        s = lax.axis_index("s")
        r0 = (c * NS + s) * cpt
        base = s * STRIPE
        @pl.when(c == 0)
        def _():
            pltpu.sync_copy(zs_hbm.at[pl.ds(base, STRIPE)], acc.at[pl.ds(base, STRIPE)])

        @pl.when(c == 1)
        def _():
            _zero_rows(rows0, CH2, 32)
            _zero_stripe(acc, base, rows0)

        plsc.subcore_barrier()
        _scatter_loop2(zs_hbm, edges_hbm, r0, ib, (rows0, rows1),
                       sems[0:2], sems[2:4], sems[4:6], acc, cpt)
        plsc.subcore_barrier()

        @pl.when(c == 0)
        def _():
            pltpu.sync_copy(acc.at[pl.ds(base, STRIPE)], u0_hbm.at[pl.ds(base, STRIPE)])

        @pl.when(c == 1)
        def _():
            pltpu.sync_copy(acc.at[pl.ds(base, STRIPE)], u1_hbm.at[pl.ds(base, STRIPE)])

    f = pl.kernel(
        body,
        out_type=[jax.ShapeDtypeStruct((NP, 32), jnp.float32),
                  jax.ShapeDtypeStruct((NP, 32), jnp.float32)],
        mesh=_mesh(),
        compiler_params=pltpu.CompilerParams(use_tc_tiling_on_sc=False),
        scratch_types=_PROP_SCRATCH,
    )
    return f(edges3d, zs)


def _row_spec(w):
    return pl.BlockSpec((R, w), lambda i: (i, 0))


def _full_spec(a, b):
    return pl.BlockSpec((a, b), lambda i: (0, 0))


def _tc1(x2, d0, d1, embed, W1):
    def body(x_ref, d0_ref, d1_ref, emb_ref, w1_ref, lo_ref, hi_ref, dv_ref):
        deg = d0_ref[:, 0:1] + d1_ref[:, 0:1] + 1.0
        dv = lax.rsqrt(deg)
        iota = lax.broadcasted_iota(jnp.int32, (R, NUM_TYPES), 1)
        oh = (x_ref[...] == iota).astype(jnp.float32)
        ew = jnp.dot(emb_ref[...], w1_ref[...], preferred_element_type=jnp.float32)
        zs = dv * jnp.dot(oh, ew, preferred_element_type=jnp.float32)
        lo_ref[...] = zs[:, :32]
        hi_ref[...] = zs[:, 32:]
        dv_ref[...] = dv

    return pl.pallas_call(
        body,
        grid=(G,),
        in_specs=[_row_spec(1), _row_spec(16), _row_spec(16),
                  _full_spec(NUM_TYPES, H), _full_spec(H, H)],
        out_specs=[_row_spec(32), _row_spec(32), _row_spec(1)],
        out_shape=[jax.ShapeDtypeStruct((NP, 32), jnp.float32),
                   jax.ShapeDtypeStruct((NP, 32), jnp.float32),
                   jax.ShapeDtypeStruct((NP, 1), jnp.float32)],
    )(x2, d0, d1, embed, W1)


def _tc_mid64(ulo, uhi, dv, W, b, out_w):
    def body(ulo_ref, uhi_ref, dv_ref, w_ref, b_ref, *outs):
        dvb = dv_ref[...]
        u = jnp.concatenate([ulo_ref[...], uhi_ref[...]], axis=1)
        h = jnp.maximum(dvb * u + b_ref[...], 0.0)
        zn = dvb * jnp.dot(h, w_ref[...], preferred_element_type=jnp.float32)
        if out_w == H:
            outs[0][...] = zn[:, :32]
            outs[1][...] = zn[:, 32:]
        else:
            outs[0][...] = zn

    if out_w == H:
        out_specs = [_row_spec(32), _row_spec(32)]
        out_shape = [jax.ShapeDtypeStruct((NP, 32), jnp.float32)] * 2
    else:
        out_specs = [_row_spec(32)]
        out_shape = [jax.ShapeDtypeStruct((NP, 32), jnp.float32)]
    return pl.pallas_call(
        body,
        grid=(G,),
        in_specs=[_row_spec(32)] * 2 + [_row_spec(1),
                  _full_spec(H, out_w), _full_spec(1, H)],
        out_specs=out_specs,
        out_shape=out_shape,
    )(ulo, uhi, dv, W, b)


def _tc4(u0, u1, dv, b3):
    def body(u0_ref, u1_ref, dv_ref, b_ref, out_ref):
        dvb = dv_ref[...]
        h = jnp.maximum(dvb * (u0_ref[...] + u1_ref[...]) + b_ref[...], 0.0)
        out_ref[...] = dvb * h

    return pl.pallas_call(
        body,
        grid=(G,),
        in_specs=[_row_spec(32)] * 2 + [_row_spec(1), _full_spec(1, OUT)],
        out_specs=[_row_spec(32)],
        out_shape=[jax.ShapeDtypeStruct((NP, 32), jnp.float32)],
    )(u0, u1, dv, b3)


def _tc5(u0, u1, dv, Wmu, bmu, Wls, bls):
    def body(u0_ref, u1_ref, dv_ref, wmu_ref, bmu_ref, wls_ref, bls_ref,
             mu_ref, ls_ref):
        t = dv_ref[...] * (u0_ref[...] + u1_ref[...])
        mu_ref[...] = jnp.dot(t, wmu_ref[...], preferred_element_type=jnp.float32) + bmu_ref[...]
        ls_ref[...] = jnp.dot(t, wls_ref[...], preferred_element_type=jnp.float32) + bls_ref[...]

    return pl.pallas_call(
        body,
        grid=(G,),
        in_specs=[_row_spec(32)] * 2 + [_row_spec(1),
                  _full_spec(OUT, OUT), _full_spec(1, OUT),
                  _full_spec(OUT, OUT), _full_spec(1, OUT)],
        out_specs=[_row_spec(32), _row_spec(32)],
        out_shape=[jax.ShapeDtypeStruct((NP, 32), jnp.float32)] * 2,
    )(u0, u1, dv, Wmu, bmu, Wls, bls)


def kernel(x, edge_index, embed, W1, b1, W2, b2, W3, b3, Wmu, bmu, Wls, bls):
    ei = edge_index.astype(jnp.int32)
    pad = jnp.full((2, EP - E), N, dtype=jnp.int32)
    ei = jnp.concatenate([ei, pad], axis=1)
    dst2d = ei[1].reshape(NROWS, CH)
    edges3d = jnp.stack([ei[0].reshape(NROWS2, CH2), ei[1].reshape(NROWS2, CH2)],
                        axis=1)
    x2 = jnp.pad(x.astype(jnp.int32), (0, NP - N)).reshape(NP, 1)

    d0, d1 = _deg_kernel(dst2d)
    zs1_lo, zs1_hi, dv = _tc1(x2, d0, d1, embed, W1)
    u1_lo, u1_hi = _prop_feat_split(edges3d, zs1_lo, zs1_hi)
    zs2_lo, zs2_hi = _tc_mid64(u1_lo, u1_hi, dv, W2, b1.reshape(1, H), H)
    u2_lo, u2_hi = _prop_feat_split(edges3d, zs2_lo, zs2_hi)
    (zs3,) = _tc_mid64(u2_lo, u2_hi, dv, W3, b2.reshape(1, H), OUT)
    u3a, u3b = _prop_edge_split(edges3d, zs3)
    (zs4,) = _tc4(u3a, u3b, dv, b3.reshape(1, OUT))
    u4a, u4b = _prop_edge_split(edges3d, zs4)
    mu, ls = _tc5(u4a, u4b, dv, Wmu, bmu.reshape(1, OUT),
                  Wls, bls.reshape(1, OUT))
    return (mu[:N], ls[:N])


# speedup vs baseline: 1.1409x; 1.1409x over previous
import functools

import jax
import jax.numpy as jnp
from jax import lax
from jax.experimental import pallas as pl
from jax.experimental.pallas import tpu as pltpu
from jax.experimental.pallas import tpu_sc as plsc

N = 50000
E = 800000
NUM_TYPES = 28
H = 64
OUT = 32
NC, NS, L = 2, 16, 16
NP = 50176
CH = 128
NROWS = 6400
EP = NROWS * CH
CH2 = 256
NROWS2 = EP // CH2
STRIPE = NP // NS
R = 512
G = NP // R


def _mesh():
    return plsc.VectorSubcoreMesh(core_axis_name="c", subcore_axis_name="s",
                                  num_cores=NC, num_subcores=NS)


def _zero_rows(buf, nrows, width):
    z = jnp.zeros((16,), jnp.float32)

    def body(i, _):
        for k in range(width // 16):
            buf[i, pl.ds(k * 16, 16)] = z
        return 0

    lax.fori_loop(0, nrows, body, 0)


def _zero_stripe(acc, base, zbuf):
    nz = zbuf.shape[0]
    nfull = STRIPE // nz
    rem = STRIPE - nfull * nz
    for k in range(nfull):
        pltpu.sync_copy(zbuf, acc.at[pl.ds(base + k * nz, nz)])
    if rem:
        pltpu.sync_copy(zbuf.at[pl.ds(0, rem)], acc.at[pl.ds(base + nfull * nz, rem)])


def _scatter_loop(tab, edges, r0, ib, rows, isems, gsems, ssems, acc, nchunks):
    def start_idx(j, k):
        pltpu.async_copy(edges.at[r0 + j], ib.at[k % 8], isems[k % 4])

    def wait_idx(j, k):
        pltpu.make_async_copy(edges.at[r0 + j], ib.at[k % 8], isems[k % 4]).wait()

    def start_gather(j, k):
        pltpu.async_copy(tab.at[ib.at[k % 8, 0]], rows[k % 4], gsems[k % 4])

    def wait_gather(j, k):
        pltpu.make_async_copy(tab.at[ib.at[k % 8, 0]], rows[k % 4],
                              gsems[k % 4]).wait()

    def start_scatter(j, k):
        pltpu.async_copy(rows[k % 4], acc.at[ib.at[k % 8, 1]], ssems[k % 4],
                         add=True)

    def wait_scatter(j, k):
        pltpu.make_async_copy(rows[k % 4], acc.at[ib.at[k % 8, 1]],
                              ssems[k % 4]).wait()

    def slot(j, k, first):
        wait_gather(j, k)
        start_scatter(j, k)
        if not first or k >= 2:
            wait_scatter(j - 2, k - 2)
        start_idx(j + 4, k + 4)
        wait_idx(j + 2, k + 2)
        start_gather(j + 2, k + 2)

    for j in range(4):
        start_idx(j, j)
    for j in range(2):
        wait_idx(j, j)
        start_gather(j, j)
    for k in range(8):
        slot(k, k, True)

    def it(i, _):
        for k in range(8):
            slot(8 * i + k, k, False)
        return 0

    nblk = nchunks // 8
    lax.fori_loop(1, nblk - 1, it, 0)
    for k in range(8):
        j = nchunks - 8 + k
        wait_gather(j, k)
        start_scatter(j, k)
        wait_scatter(j - 2, k - 2)
        if k < 4:
            start_idx(j + 4, k + 4)
        if k < 6:
            wait_idx(j + 2, k + 2)
            start_gather(j + 2, k + 2)
    wait_scatter(nchunks - 2, nchunks - 2)
    wait_scatter(nchunks - 1, nchunks - 1)


def _deg_kernel(dst2d):
    cpt = NROWS // (NC * NS)

    def body(dst_hbm, d0_hbm, d1_hbm, didx, ones, zbuf, acc):
        c = lax.axis_index("c")
        s = lax.axis_index("s")
        r0 = (c * NS + s) * cpt
        pltpu.sync_copy(dst_hbm.at[pl.ds(r0, cpt)], didx)
        _zero_rows(zbuf, CH, 16)
        base = s * STRIPE
        _zero_stripe(acc, base, zbuf)
        _zero_rows(ones, CH, 16)

        def setones(i, _):
            o = jnp.full((16,), 1.0, jnp.float32)
            ones[i, pl.ds(0, 16)] = o
            return 0

        lax.fori_loop(0, CH, setones, 0)
        plsc.subcore_barrier()

        def it(j, _):
            pltpu.sync_copy(ones, acc.at[didx.at[j]], add=True)
            return 0

        lax.fori_loop(0, cpt, it, 0)
        plsc.subcore_barrier()

        @pl.when(c == 0)
        def _():
            pltpu.sync_copy(acc.at[pl.ds(base, STRIPE)], d0_hbm.at[pl.ds(base, STRIPE)])

        @pl.when(c == 1)
        def _():
            pltpu.sync_copy(acc.at[pl.ds(base, STRIPE)], d1_hbm.at[pl.ds(base, STRIPE)])

    f = pl.kernel(
        body,
        out_type=[jax.ShapeDtypeStruct((NP, 16), jnp.float32),
                  jax.ShapeDtypeStruct((NP, 16), jnp.float32)],
        mesh=_mesh(),
        compiler_params=pltpu.CompilerParams(use_tc_tiling_on_sc=False),
        scratch_types=[
            pltpu.VMEM((cpt, CH), jnp.int32),
            pltpu.VMEM((CH, 16), jnp.float32),
            pltpu.VMEM((CH, 16), jnp.float32),
            pltpu.VMEM_SHARED((NP, 16), jnp.float32),
        ],
    )
    return f(dst2d)


_PROP_SCRATCH = [
    pltpu.VMEM((8, 2, CH), jnp.int32),
    pltpu.VMEM((CH, 32), jnp.float32),
    pltpu.VMEM((CH, 32), jnp.float32),
    pltpu.VMEM((CH, 32), jnp.float32),
    pltpu.VMEM((CH, 32), jnp.float32),
    pltpu.VMEM_SHARED((NP, 32), jnp.float32),
] + [pltpu.SemaphoreType.DMA] * 12


def _prop_feat_split(edges3d, zs_lo, zs_hi):
    cpt = NROWS // NS

    def body(edges_hbm, lo_hbm, hi_hbm, ulo_hbm, uhi_hbm,
             ib, rows0, rows1, rows2, rows3, acc, *sems):
        c = lax.axis_index("c")
        s = lax.axis_index("s")
        r0 = s * cpt
        base = s * STRIPE
        @pl.when(c == 0)
        def _():
            pltpu.sync_copy(lo_hbm.at[pl.ds(base, STRIPE)], acc.at[pl.ds(base, STRIPE)])

        @pl.when(c == 1)
        def _():
            pltpu.sync_copy(hi_hbm.at[pl.ds(base, STRIPE)], acc.at[pl.ds(base, STRIPE)])

        plsc.subcore_barrier()
        rows = (rows0, rows1, rows2, rows3)
        isems, gsems, ssems = sems[0:4], sems[4:8], sems[8:12]

        @pl.when(c == 0)
        def _():
            _scatter_loop(lo_hbm, edges_hbm, r0, ib, rows, isems, gsems,
                          ssems, acc, cpt)

        @pl.when(c == 1)
        def _():
            _scatter_loop(hi_hbm, edges_hbm, r0, ib, rows, isems, gsems,
                          ssems, acc, cpt)

        plsc.subcore_barrier()

        @pl.when(c == 0)
        def _():
            pltpu.sync_copy(acc.at[pl.ds(base, STRIPE)], ulo_hbm.at[pl.ds(base, STRIPE)])

        @pl.when(c == 1)
        def _():
            pltpu.sync_copy(acc.at[pl.ds(base, STRIPE)], uhi_hbm.at[pl.ds(base, STRIPE)])

    f = pl.kernel(
        body,
        out_type=[jax.ShapeDtypeStruct((NP, 32), jnp.float32),
                  jax.ShapeDtypeStruct((NP, 32), jnp.float32)],
        mesh=_mesh(),
        compiler_params=pltpu.CompilerParams(use_tc_tiling_on_sc=False),
        scratch_types=_PROP_SCRATCH,
    )
    return f(edges3d, zs_lo, zs_hi)


def _prop_edge_split(edges3d, zs):
    cpt = NROWS // (NC * NS)

    def body(edges_hbm, zs_hbm, u0_hbm, u1_hbm,
             ib, rows0, rows1, rows2, rows3, acc, *sems):
        c = lax.axis_index("c")
        s = lax.axis_index("s")
        r0 = (s * NC + c) * cpt
        base = s * STRIPE
        @pl.when(c == 0)
        def _():
            pltpu.sync_copy(zs_hbm.at[pl.ds(base, STRIPE)], acc.at[pl.ds(base, STRIPE)])

        @pl.when(c == 1)
        def _():
            _zero_rows(rows0, CH, 32)
            _zero_stripe(acc, base, rows0)

        plsc.subcore_barrier()
        _scatter_loop(zs_hbm, edges_hbm, r0, ib, (rows0, rows1, rows2, rows3),
                      sems[0:4], sems[4:8], sems[8:12], acc, cpt)
        plsc.subcore_barrier()

        @pl.when(c == 0)
        def _():
            pltpu.sync_copy(acc.at[pl.ds(base, STRIPE)], u0_hbm.at[pl.ds(base, STRIPE)])

        @pl.when(c == 1)
        def _():
            pltpu.sync_copy(acc.at[pl.ds(base, STRIPE)], u1_hbm.at[pl.ds(base, STRIPE)])

    f = pl.kernel(
        body,
        out_type=[jax.ShapeDtypeStruct((NP, 32), jnp.float32),
                  jax.ShapeDtypeStruct((NP, 32), jnp.float32)],
        mesh=_mesh(),
        compiler_params=pltpu.CompilerParams(use_tc_tiling_on_sc=False),
        scratch_types=_PROP_SCRATCH,
    )
    return f(edges3d, zs)


def _row_spec(w):
    return pl.BlockSpec((R, w), lambda i: (i, 0))


def _full_spec(a, b):
    return pl.BlockSpec((a, b), lambda i: (0, 0))


def _tc1(x2, d0, d1, embed, W1):
    def body(x_ref, d0_ref, d1_ref, emb_ref, w1_ref, lo_ref, hi_ref, dv_ref):
        deg = d0_ref[:, 0:1] + d1_ref[:, 0:1] + 1.0
        dv = lax.rsqrt(deg)
        iota = lax.broadcasted_iota(jnp.int32, (R, NUM_TYPES), 1)
        oh = (x_ref[...] == iota).astype(jnp.float32)
        ew = jnp.dot(emb_ref[...], w1_ref[...], preferred_element_type=jnp.float32)
        zs = dv * jnp.dot(oh, ew, preferred_element_type=jnp.float32)
        lo_ref[...] = zs[:, :32]
        hi_ref[...] = zs[:, 32:]
        dv_ref[...] = dv

    return pl.pallas_call(
        body,
        grid=(G,),
        in_specs=[_row_spec(1), _row_spec(16), _row_spec(16),
                  _full_spec(NUM_TYPES, H), _full_spec(H, H)],
        out_specs=[_row_spec(32), _row_spec(32), _row_spec(1)],
        out_shape=[jax.ShapeDtypeStruct((NP, 32), jnp.float32),
                   jax.ShapeDtypeStruct((NP, 32), jnp.float32),
                   jax.ShapeDtypeStruct((NP, 1), jnp.float32)],
    )(x2, d0, d1, embed, W1)


def _tc_mid64(ulo, uhi, dv, W, b, out_w):
    def body(ulo_ref, uhi_ref, dv_ref, w_ref, b_ref, *outs):
        dvb = dv_ref[...]
        u = jnp.concatenate([ulo_ref[...], uhi_ref[...]], axis=1)
        h = jnp.maximum(dvb * u + b_ref[...], 0.0)
        zn = dvb * jnp.dot(h, w_ref[...], preferred_element_type=jnp.float32)
        if out_w == H:
            outs[0][...] = zn[:, :32]
            outs[1][...] = zn[:, 32:]
        else:
            outs[0][...] = zn

    if out_w == H:
        out_specs = [_row_spec(32), _row_spec(32)]
        out_shape = [jax.ShapeDtypeStruct((NP, 32), jnp.float32)] * 2
    else:
        out_specs = [_row_spec(32)]
        out_shape = [jax.ShapeDtypeStruct((NP, 32), jnp.float32)]
    return pl.pallas_call(
        body,
        grid=(G,),
        in_specs=[_row_spec(32)] * 2 + [_row_spec(1),
                  _full_spec(H, out_w), _full_spec(1, H)],
        out_specs=out_specs,
        out_shape=out_shape,
    )(ulo, uhi, dv, W, b)


def _tc4(u0, u1, dv, b3):
    def body(u0_ref, u1_ref, dv_ref, b_ref, out_ref):
        dvb = dv_ref[...]
        h = jnp.maximum(dvb * (u0_ref[...] + u1_ref[...]) + b_ref[...], 0.0)
        out_ref[...] = dvb * h

    return pl.pallas_call(
        body,
        grid=(G,),
        in_specs=[_row_spec(32)] * 2 + [_row_spec(1), _full_spec(1, OUT)],
        out_specs=[_row_spec(32)],
        out_shape=[jax.ShapeDtypeStruct((NP, 32), jnp.float32)],
    )(u0, u1, dv, b3)


def _tc5(u0, u1, dv, Wmu, bmu, Wls, bls):
    def body(u0_ref, u1_ref, dv_ref, wmu_ref, bmu_ref, wls_ref, bls_ref,
             mu_ref, ls_ref):
        t = dv_ref[...] * (u0_ref[...] + u1_ref[...])
        mu_ref[...] = jnp.dot(t, wmu_ref[...], preferred_element_type=jnp.float32) + bmu_ref[...]
        ls_ref[...] = jnp.dot(t, wls_ref[...], preferred_element_type=jnp.float32) + bls_ref[...]

    return pl.pallas_call(
        body,
        grid=(G,),
        in_specs=[_row_spec(32)] * 2 + [_row_spec(1),
                  _full_spec(OUT, OUT), _full_spec(1, OUT),
                  _full_spec(OUT, OUT), _full_spec(1, OUT)],
        out_specs=[_row_spec(32), _row_spec(32)],
        out_shape=[jax.ShapeDtypeStruct((NP, 32), jnp.float32)] * 2,
    )(u0, u1, dv, Wmu, bmu, Wls, bls)


def kernel(x, edge_index, embed, W1, b1, W2, b2, W3, b3, Wmu, bmu, Wls, bls):
    ei = edge_index.astype(jnp.int32)
    pad = jnp.full((2, EP - E), N, dtype=jnp.int32)
    ei = jnp.concatenate([ei, pad], axis=1)
    dst2d = ei[1].reshape(NROWS, CH)
    edges3d = jnp.stack([ei[0].reshape(NROWS, CH), ei[1].reshape(NROWS, CH)],
                        axis=1)
    x2 = jnp.pad(x.astype(jnp.int32), (0, NP - N)).reshape(NP, 1)

    d0, d1 = _deg_kernel(dst2d)
    zs1_lo, zs1_hi, dv = _tc1(x2, d0, d1, embed, W1)
    u1_lo, u1_hi = _prop_feat_split(edges3d, zs1_lo, zs1_hi)
    zs2_lo, zs2_hi = _tc_mid64(u1_lo, u1_hi, dv, W2, b1.reshape(1, H), H)
    u2_lo, u2_hi = _prop_feat_split(edges3d, zs2_lo, zs2_hi)
    (zs3,) = _tc_mid64(u2_lo, u2_hi, dv, W3, b2.reshape(1, H), OUT)
    u3a, u3b = _prop_edge_split(edges3d, zs3)
    (zs4,) = _tc4(u3a, u3b, dv, b3.reshape(1, OUT))
    u4a, u4b = _prop_edge_split(edges3d, zs4)
    mu, ls = _tc5(u4a, u4b, dv, Wmu, bmu.reshape(1, OUT),
                  Wls, bls.reshape(1, OUT))
    return (mu[:N], ls[:N])


# speedup vs baseline: 1.1518x; 1.0095x over previous
import functools

import jax
import jax.numpy as jnp
from jax import lax
from jax.experimental import pallas as pl
from jax.experimental.pallas import tpu as pltpu
from jax.experimental.pallas import tpu_sc as plsc

N = 50000
E = 800000
NUM_TYPES = 28
H = 64
OUT = 32
NC, NS, L = 2, 16, 16
NP = 50176
CH = 128
NROWS = 6400
EP = NROWS * CH
CH2 = 256
NROWS2 = EP // CH2
STRIPE = NP // NS
R = 512
G = NP // R


def _mesh():
    return plsc.VectorSubcoreMesh(core_axis_name="c", subcore_axis_name="s",
                                  num_cores=NC, num_subcores=NS)


def _zero_rows(buf, nrows, width):
    z = jnp.zeros((16,), jnp.float32)

    def body(i, _):
        for k in range(width // 16):
            buf[i, pl.ds(k * 16, 16)] = z
        return 0

    lax.fori_loop(0, nrows, body, 0)


def _zero_stripe(acc, base, zbuf):
    nz = zbuf.shape[0]
    nfull = STRIPE // nz
    rem = STRIPE - nfull * nz
    for k in range(nfull):
        pltpu.sync_copy(zbuf, acc.at[pl.ds(base + k * nz, nz)])
    if rem:
        pltpu.sync_copy(zbuf.at[pl.ds(0, rem)], acc.at[pl.ds(base + nfull * nz, rem)])


def _scatter_loop(tab, edges, r0, ib, rows, isems, gsems, ssems, acc, nchunks):
    def start_idx(j, k):
        pltpu.async_copy(edges.at[r0 + j], ib.at[k % 8], isems[k % 4])

    def wait_idx(j, k):
        pltpu.make_async_copy(edges.at[r0 + j], ib.at[k % 8], isems[k % 4]).wait()

    def start_gather(j, k):
        pltpu.async_copy(tab.at[ib.at[k % 8, 0]], rows[k % 4], gsems[k % 4])

    def wait_gather(j, k):
        pltpu.make_async_copy(tab.at[ib.at[k % 8, 0]], rows[k % 4],
                              gsems[k % 4]).wait()

    def start_scatter(j, k):
        pltpu.async_copy(rows[k % 4], acc.at[ib.at[k % 8, 1]], ssems[k % 4],
                         add=True)

    def wait_scatter(j, k):
        pltpu.make_async_copy(rows[k % 4], acc.at[ib.at[k % 8, 1]],
                              ssems[k % 4]).wait()

    def slot(j, k, first):
        wait_gather(j, k)
        start_scatter(j, k)
        if not first or k >= 2:
            wait_scatter(j - 2, k - 2)
        start_idx(j + 4, k + 4)
        wait_idx(j + 2, k + 2)
        start_gather(j + 2, k + 2)

    for j in range(4):
        start_idx(j, j)
    for j in range(2):
        wait_idx(j, j)
        start_gather(j, j)
    for k in range(8):
        slot(k, k, True)

    def it(i, _):
        for k in range(8):
            slot(8 * i + k, k, False)
        return 0

    nblk = nchunks // 8
    lax.fori_loop(1, nblk - 1, it, 0)
    for k in range(8):
        j = nchunks - 8 + k
        wait_gather(j, k)
        start_scatter(j, k)
        wait_scatter(j - 2, k - 2)
        if k < 4:
            start_idx(j + 4, k + 4)
        if k < 6:
            wait_idx(j + 2, k + 2)
            start_gather(j + 2, k + 2)
    wait_scatter(nchunks - 2, nchunks - 2)
    wait_scatter(nchunks - 1, nchunks - 1)


def _deg_kernel(dst2d):
    cpt = NROWS // (NC * NS)

    def body(dst_hbm, d0_hbm, d1_hbm, didx, ones, zbuf, acc):
        c = lax.axis_index("c")
        s = lax.axis_index("s")
        r0 = (c * NS + s) * cpt
        pltpu.sync_copy(dst_hbm.at[pl.ds(r0, cpt)], didx)
        _zero_rows(zbuf, CH, 16)
        base = s * STRIPE
        _zero_stripe(acc, base, zbuf)
        _zero_rows(ones, CH, 16)

        def setones(i, _):
            o = jnp.full((16,), 1.0, jnp.float32)
            ones[i, pl.ds(0, 16)] = o
            return 0

        lax.fori_loop(0, CH, setones, 0)
        plsc.subcore_barrier()

        def it(j, _):
            pltpu.sync_copy(ones, acc.at[didx.at[j]], add=True)
            return 0

        lax.fori_loop(0, cpt, it, 0)
        plsc.subcore_barrier()

        @pl.when(c == 0)
        def _():
            pltpu.sync_copy(acc.at[pl.ds(base, STRIPE)], d0_hbm.at[pl.ds(base, STRIPE)])

        @pl.when(c == 1)
        def _():
            pltpu.sync_copy(acc.at[pl.ds(base, STRIPE)], d1_hbm.at[pl.ds(base, STRIPE)])

    f = pl.kernel(
        body,
        out_type=[jax.ShapeDtypeStruct((NP, 16), jnp.float32),
                  jax.ShapeDtypeStruct((NP, 16), jnp.float32)],
        mesh=_mesh(),
        compiler_params=pltpu.CompilerParams(use_tc_tiling_on_sc=False),
        scratch_types=[
            pltpu.VMEM((cpt, CH), jnp.int32),
            pltpu.VMEM((CH, 16), jnp.float32),
            pltpu.VMEM((CH, 16), jnp.float32),
            pltpu.VMEM_SHARED((NP, 16), jnp.float32),
        ],
    )
    return f(dst2d)


_PROP_SCRATCH = [
    pltpu.VMEM((8, 2, CH), jnp.int32),
    pltpu.VMEM((CH, 32), jnp.float32),
    pltpu.VMEM((CH, 32), jnp.float32),
    pltpu.VMEM((CH, 32), jnp.float32),
    pltpu.VMEM((CH, 32), jnp.float32),
    pltpu.VMEM_SHARED((NP, 32), jnp.float32),
] + [pltpu.SemaphoreType.DMA] * 12


def _prop_feat_split(edges3d, zs_lo, zs_hi):
    cpt = NROWS // NS

    def body(edges_hbm, lo_hbm, hi_hbm, ulo_hbm, uhi_hbm,
             ib, rows0, rows1, rows2, rows3, acc, *sems):
        c = lax.axis_index("c")
        s = lax.axis_index("s")
        r0 = s * cpt
        base = s * STRIPE
        @pl.when(c == 0)
        def _():
            pltpu.sync_copy(lo_hbm.at[pl.ds(base, STRIPE)], acc.at[pl.ds(base, STRIPE)])

        @pl.when(c == 1)
        def _():
            pltpu.sync_copy(hi_hbm.at[pl.ds(base, STRIPE)], acc.at[pl.ds(base, STRIPE)])

        plsc.subcore_barrier()
        rows = (rows0, rows1, rows2, rows3)
        isems, gsems, ssems = sems[0:4], sems[4:8], sems[8:12]

        @pl.when(c == 0)
        def _():
            _scatter_loop(lo_hbm, edges_hbm, r0, ib, rows, isems, gsems,
                          ssems, acc, cpt)

        @pl.when(c == 1)
        def _():
            _scatter_loop(hi_hbm, edges_hbm, r0, ib, rows, isems, gsems,
                          ssems, acc, cpt)

        plsc.subcore_barrier()

        @pl.when(c == 0)
        def _():
            pltpu.sync_copy(acc.at[pl.ds(base, STRIPE)], ulo_hbm.at[pl.ds(base, STRIPE)])

        @pl.when(c == 1)
        def _():
            pltpu.sync_copy(acc.at[pl.ds(base, STRIPE)], uhi_hbm.at[pl.ds(base, STRIPE)])

    f = pl.kernel(
        body,
        out_type=[jax.ShapeDtypeStruct((NP, 32), jnp.float32),
                  jax.ShapeDtypeStruct((NP, 32), jnp.float32)],
        mesh=_mesh(),
        compiler_params=pltpu.CompilerParams(use_tc_tiling_on_sc=False),
        scratch_types=_PROP_SCRATCH,
    )
    return f(edges3d, zs_lo, zs_hi)


def _prop_edge_split(edges3d, zs):
    cpt = NROWS // (NC * NS)

    def body(edges_hbm, zs_hbm, u0_hbm, u1_hbm,
             ib, rows0, rows1, rows2, rows3, acc, *sems):
        c = lax.axis_index("c")
        s = lax.axis_index("s")
        r0 = (s * NC + c) * cpt
        base = s * STRIPE
        @pl.when(c == 0)
        def _():
            pltpu.sync_copy(zs_hbm.at[pl.ds(base, STRIPE)], acc.at[pl.ds(base, STRIPE)])

        @pl.when(c == 1)
        def _():
            _zero_rows(rows0, CH, 32)
            _zero_stripe(acc, base, rows0)

        plsc.subcore_barrier()
        _scatter_loop(zs_hbm, edges_hbm, r0, ib, (rows0, rows1, rows2, rows3),
                      sems[0:4], sems[4:8], sems[8:12], acc, cpt)
        plsc.subcore_barrier()

        @pl.when(c == 0)
        def _():
            pltpu.sync_copy(acc.at[pl.ds(base, STRIPE)], u0_hbm.at[pl.ds(base, STRIPE)])

        @pl.when(c == 1)
        def _():
            pltpu.sync_copy(acc.at[pl.ds(base, STRIPE)], u1_hbm.at[pl.ds(base, STRIPE)])

    f = pl.kernel(
        body,
        out_type=[jax.ShapeDtypeStruct((NP, 32), jnp.float32),
                  jax.ShapeDtypeStruct((NP, 32), jnp.float32)],
        mesh=_mesh(),
        compiler_params=pltpu.CompilerParams(use_tc_tiling_on_sc=False),
        scratch_types=_PROP_SCRATCH,
    )
    return f(edges3d, zs)


def _prop4_fused(edges3d, u3a, u3b, dv1d, b3):
    cpt = NROWS // (NC * NS)

    def body(edges_hbm, ua_hbm, ub_hbm, dv_hbm, b_hbm, z0_hbm, z1_hbm,
             t0_hbm, t1_hbm, ib, rows0, rows1, rows2, rows3, acc, bb, *sems):
        va, vb, dvb = rows1, rows2, rows3

        c = lax.axis_index("c")
        s = lax.axis_index("s")
        r0 = (s * NC + c) * cpt
        base = s * STRIPE
        pltpu.sync_copy(b_hbm, bb)
        b0 = bb[pl.ds(0, 16)]
        b1 = bb[pl.ds(16, 16)]
        nchk = STRIPE // CH
        rem = STRIPE - nchk * CH

        def fill(off, nrows):
            pltpu.sync_copy(ua_hbm.at[pl.ds(off, nrows)], va.at[pl.ds(0, nrows)])
            pltpu.sync_copy(ub_hbm.at[pl.ds(off, nrows)], vb.at[pl.ds(0, nrows)])
            pltpu.sync_copy(dv_hbm.at[pl.ds(off, nrows)], dvb.at[pl.ds(0, nrows)])

            def rowfn(r, _):
                d0 = dvb[r, pl.ds(0, 16)]
                d1 = dvb[r, pl.ds(16, 16)]
                x0 = jnp.maximum((va[r, pl.ds(0, 16)] + vb[r, pl.ds(0, 16)]) * d0 + b0, 0.0)
                x1 = jnp.maximum((va[r, pl.ds(16, 16)] + vb[r, pl.ds(16, 16)]) * d1 + b1, 0.0)
                va[r, pl.ds(0, 16)] = x0 * d0
                va[r, pl.ds(16, 16)] = x1 * d1
                return 0

            lax.fori_loop(0, nrows, rowfn, 0)

            @pl.when(c == 0)
            def _():
                pltpu.sync_copy(va.at[pl.ds(0, nrows)], z0_hbm.at[pl.ds(off, nrows)])
                pltpu.sync_copy(va.at[pl.ds(0, nrows)], acc.at[pl.ds(off, nrows)])

            @pl.when(c == 1)
            def _():
                pltpu.sync_copy(va.at[pl.ds(0, nrows)], z1_hbm.at[pl.ds(off, nrows)])

        for k in range(nchk):
            fill(base + k * CH, CH)
        fill(base + nchk * CH, rem)

        @pl.when(c == 1)
        def _():
            _zero_rows(rows0, CH, 32)
            _zero_stripe(acc, base, rows0)

        plsc.subcore_barrier()
        rows = (rows0, rows1, rows2, rows3)

        @pl.when(c == 0)
        def _():
            _scatter_loop(z0_hbm, edges_hbm, r0, ib, rows, sems[0:4],
                          sems[4:8], sems[8:12], acc, cpt)

        @pl.when(c == 1)
        def _():
            _scatter_loop(z1_hbm, edges_hbm, r0, ib, rows, sems[0:4],
                          sems[4:8], sems[8:12], acc, cpt)

        plsc.subcore_barrier()

        @pl.when(c == 0)
        def _():
            pltpu.sync_copy(acc.at[pl.ds(base, STRIPE)], t0_hbm.at[pl.ds(base, STRIPE)])

        @pl.when(c == 1)
        def _():
            pltpu.sync_copy(acc.at[pl.ds(base, STRIPE)], t1_hbm.at[pl.ds(base, STRIPE)])

    f = pl.kernel(
        body,
        out_type=[jax.ShapeDtypeStruct((NP, 32), jnp.float32)] * 4,
        mesh=_mesh(),
        compiler_params=pltpu.CompilerParams(use_tc_tiling_on_sc=False),
        scratch_types=_PROP_SCRATCH[:6] + [
            pltpu.VMEM((32,), jnp.float32),
        ] + [pltpu.SemaphoreType.DMA] * 12,
    )
    z0, z1, t0, t1 = f(edges3d, u3a, u3b, dv1d, b3)
    return t0, t1


def _row_spec(w):
    return pl.BlockSpec((R, w), lambda i: (i, 0))


def _full_spec(a, b):
    return pl.BlockSpec((a, b), lambda i: (0, 0))


def _tc1(x2, d0, d1, embed, W1):
    def body(x_ref, d0_ref, d1_ref, emb_ref, w1_ref, lo_ref, hi_ref, dv_ref,
             dvr_ref):
        deg = d0_ref[:, 0:1] + d1_ref[:, 0:1] + 1.0
        dv = lax.rsqrt(deg)
        iota = lax.broadcasted_iota(jnp.int32, (R, NUM_TYPES), 1)
        oh = (x_ref[...] == iota).astype(jnp.float32)
        ew = jnp.dot(emb_ref[...], w1_ref[...], preferred_element_type=jnp.float32)
        zs = dv * jnp.dot(oh, ew, preferred_element_type=jnp.float32)
        lo_ref[...] = zs[:, :32]
        hi_ref[...] = zs[:, 32:]
        dv_ref[...] = dv
        dvr_ref[...] = jnp.broadcast_to(dv, (R, 32))

    return pl.pallas_call(
        body,
        grid=(G,),
        in_specs=[_row_spec(1), _row_spec(16), _row_spec(16),
                  _full_spec(NUM_TYPES, H), _full_spec(H, H)],
        out_specs=[_row_spec(32), _row_spec(32), _row_spec(1), _row_spec(32)],
        out_shape=[jax.ShapeDtypeStruct((NP, 32), jnp.float32),
                   jax.ShapeDtypeStruct((NP, 32), jnp.float32),
                   jax.ShapeDtypeStruct((NP, 1), jnp.float32),
                   jax.ShapeDtypeStruct((NP, 32), jnp.float32)],
    )(x2, d0, d1, embed, W1)


def _tc_mid64(ulo, uhi, dv, W, b, out_w):
    def body(ulo_ref, uhi_ref, dv_ref, w_ref, b_ref, *outs):
        dvb = dv_ref[...]
        u = jnp.concatenate([ulo_ref[...], uhi_ref[...]], axis=1)
        h = jnp.maximum(dvb * u + b_ref[...], 0.0)
        zn = dvb * jnp.dot(h, w_ref[...], preferred_element_type=jnp.float32)
        if out_w == H:
            outs[0][...] = zn[:, :32]
            outs[1][...] = zn[:, 32:]
        else:
            outs[0][...] = zn

    if out_w == H:
        out_specs = [_row_spec(32), _row_spec(32)]
        out_shape = [jax.ShapeDtypeStruct((NP, 32), jnp.float32)] * 2
    else:
        out_specs = [_row_spec(32)]
        out_shape = [jax.ShapeDtypeStruct((NP, 32), jnp.float32)]
    return pl.pallas_call(
        body,
        grid=(G,),
        in_specs=[_row_spec(32)] * 2 + [_row_spec(1),
                  _full_spec(H, out_w), _full_spec(1, H)],
        out_specs=out_specs,
        out_shape=out_shape,
    )(ulo, uhi, dv, W, b)


def _tc4(u0, u1, dv, b3):
    def body(u0_ref, u1_ref, dv_ref, b_ref, out_ref):
        dvb = dv_ref[...]
        h = jnp.maximum(dvb * (u0_ref[...] + u1_ref[...]) + b_ref[...], 0.0)
        out_ref[...] = dvb * h

    return pl.pallas_call(
        body,
        grid=(G,),
        in_specs=[_row_spec(32)] * 2 + [_row_spec(1), _full_spec(1, OUT)],
        out_specs=[_row_spec(32)],
        out_shape=[jax.ShapeDtypeStruct((NP, 32), jnp.float32)],
    )(u0, u1, dv, b3)


def _tc5(u0, u1, dv, Wmu, bmu, Wls, bls):
    def body(u0_ref, u1_ref, dv_ref, wmu_ref, bmu_ref, wls_ref, bls_ref,
             mu_ref, ls_ref):
        t = dv_ref[...] * (u0_ref[...] + u1_ref[...])
        mu_ref[...] = jnp.dot(t, wmu_ref[...], preferred_element_type=jnp.float32) + bmu_ref[...]
        ls_ref[...] = jnp.dot(t, wls_ref[...], preferred_element_type=jnp.float32) + bls_ref[...]

    return pl.pallas_call(
        body,
        grid=(G,),
        in_specs=[_row_spec(32)] * 2 + [_row_spec(1),
                  _full_spec(OUT, OUT), _full_spec(1, OUT),
                  _full_spec(OUT, OUT), _full_spec(1, OUT)],
        out_specs=[_row_spec(32), _row_spec(32)],
        out_shape=[jax.ShapeDtypeStruct((NP, 32), jnp.float32)] * 2,
    )(u0, u1, dv, Wmu, bmu, Wls, bls)


def kernel(x, edge_index, embed, W1, b1, W2, b2, W3, b3, Wmu, bmu, Wls, bls):
    ei = edge_index.astype(jnp.int32)
    pad = jnp.full((2, EP - E), N, dtype=jnp.int32)
    ei = jnp.concatenate([ei, pad], axis=1)
    dst2d = ei[1].reshape(NROWS, CH)
    edges3d = jnp.stack([ei[0].reshape(NROWS, CH), ei[1].reshape(NROWS, CH)],
                        axis=1)
    x2 = jnp.pad(x.astype(jnp.int32), (0, NP - N)).reshape(NP, 1)

    d0, d1 = _deg_kernel(dst2d)
    zs1_lo, zs1_hi, dv, dv32 = _tc1(x2, d0, d1, embed, W1)
    u1_lo, u1_hi = _prop_feat_split(edges3d, zs1_lo, zs1_hi)
    zs2_lo, zs2_hi = _tc_mid64(u1_lo, u1_hi, dv, W2, b1.reshape(1, H), H)
    u2_lo, u2_hi = _prop_feat_split(edges3d, zs2_lo, zs2_hi)
    (zs3,) = _tc_mid64(u2_lo, u2_hi, dv, W3, b2.reshape(1, H), OUT)
    u3a, u3b = _prop_edge_split(edges3d, zs3)
    u4a, u4b = _prop4_fused(edges3d, u3a, u3b, dv32, b3)
    mu, ls = _tc5(u4a, u4b, dv, Wmu, bmu.reshape(1, OUT),
                  Wls, bls.reshape(1, OUT))
    return (mu[:N], ls[:N])


# speedup vs baseline: 1.1518x; 1.0000x over previous
import jax
import jax.numpy as jnp
from jax import lax
from jax.experimental import pallas as pl
from jax.experimental.pallas import tpu as pltpu
from jax.experimental.pallas import tpu_sc as plsc

N = 50000
E = 800000
NUM_TYPES = 28
H = 64
OUT = 32
NC, NS, L = 2, 16, 16
NP = 50176
CH = 128
NROWS = 6400
EP = NROWS * CH
STRIPE = NP // NS
R = 512
G = NP // R


def _mesh():
    return plsc.VectorSubcoreMesh(core_axis_name="c", subcore_axis_name="s",
                                  num_cores=NC, num_subcores=NS)


def _zero_rows(buf, nrows, width):
    z = jnp.zeros((16,), jnp.float32)

    def body(i, _):
        for k in range(width // 16):
            buf[i, pl.ds(k * 16, 16)] = z
        return 0

    lax.fori_loop(0, nrows, body, 0)


def _zero_stripe(acc, base, zbuf):
    nz = zbuf.shape[0]
    nfull = STRIPE // nz
    rem = STRIPE - nfull * nz
    for k in range(nfull):
        pltpu.sync_copy(zbuf, acc.at[pl.ds(base + k * nz, nz)])
    if rem:
        pltpu.sync_copy(zbuf.at[pl.ds(0, rem)], acc.at[pl.ds(base + nfull * nz, rem)])


def _scatter_loop(tab, edges, r0, ib, rows, isems, gsems, ssems, acc, nchunks):
    def start_idx(j, k):
        pltpu.async_copy(edges.at[r0 + j], ib.at[k % 8], isems[k % 4])

    def wait_idx(j, k):
        pltpu.make_async_copy(edges.at[r0 + j], ib.at[k % 8], isems[k % 4]).wait()

    def start_gather(j, k):
        pltpu.async_copy(tab.at[ib.at[k % 8, 0]], rows[k % 4], gsems[k % 4])

    def wait_gather(j, k):
        pltpu.make_async_copy(tab.at[ib.at[k % 8, 0]], rows[k % 4],
                              gsems[k % 4]).wait()

    def start_scatter(j, k):
        pltpu.async_copy(rows[k % 4], acc.at[ib.at[k % 8, 1]], ssems[k % 4],
                         add=True)

    def wait_scatter(j, k):
        pltpu.make_async_copy(rows[k % 4], acc.at[ib.at[k % 8, 1]],
                              ssems[k % 4]).wait()

    def slot(j, k, first):
        wait_gather(j, k)
        start_scatter(j, k)
        if not first or k >= 2:
            wait_scatter(j - 2, k - 2)
        start_idx(j + 4, k + 4)
        wait_idx(j + 2, k + 2)
        start_gather(j + 2, k + 2)

    for j in range(4):
        start_idx(j, j)
    for j in range(2):
        wait_idx(j, j)
        start_gather(j, j)
    for k in range(8):
        slot(k, k, True)

    def it(i, _):
        for k in range(8):
            slot(8 * i + k, k, False)
        return 0

    nblk = nchunks // 8
    lax.fori_loop(1, nblk - 1, it, 0)
    for k in range(8):
        j = nchunks - 8 + k
        wait_gather(j, k)
        start_scatter(j, k)
        wait_scatter(j - 2, k - 2)
        if k < 4:
            start_idx(j + 4, k + 4)
        if k < 6:
            wait_idx(j + 2, k + 2)
            start_gather(j + 2, k + 2)
    wait_scatter(nchunks - 2, nchunks - 2)
    wait_scatter(nchunks - 1, nchunks - 1)


def _deg_kernel(dst2d):
    cpt = NROWS // (NC * NS)

    def body(dst_hbm, d0_hbm, d1_hbm, didx, ones, zbuf, acc):
        c = lax.axis_index("c")
        s = lax.axis_index("s")
        r0 = (c * NS + s) * cpt
        pltpu.sync_copy(dst_hbm.at[pl.ds(r0, cpt)], didx)
        _zero_rows(zbuf, CH, 16)
        base = s * STRIPE
        _zero_stripe(acc, base, zbuf)
        _zero_rows(ones, CH, 16)

        def setones(i, _):
            o = jnp.full((16,), 1.0, jnp.float32)
            ones[i, pl.ds(0, 16)] = o
            return 0

        lax.fori_loop(0, CH, setones, 0)
        plsc.subcore_barrier()

        def it(j, _):
            pltpu.sync_copy(ones, acc.at[didx.at[j]], add=True)
            return 0

        lax.fori_loop(0, cpt, it, 0)
        plsc.subcore_barrier()

        @pl.when(c == 0)
        def _():
            pltpu.sync_copy(acc.at[pl.ds(base, STRIPE)], d0_hbm.at[pl.ds(base, STRIPE)])

        @pl.when(c == 1)
        def _():
            pltpu.sync_copy(acc.at[pl.ds(base, STRIPE)], d1_hbm.at[pl.ds(base, STRIPE)])

    f = pl.kernel(
        body,
        out_type=[jax.ShapeDtypeStruct((NP, 16), jnp.float32),
                  jax.ShapeDtypeStruct((NP, 16), jnp.float32)],
        mesh=_mesh(),
        compiler_params=pltpu.CompilerParams(use_tc_tiling_on_sc=False),
        scratch_types=[
            pltpu.VMEM((cpt, CH), jnp.int32),
            pltpu.VMEM((CH, 16), jnp.float32),
            pltpu.VMEM((CH, 16), jnp.float32),
            pltpu.VMEM_SHARED((NP, 16), jnp.float32),
        ],
    )
    return f(dst2d)


_PROP_SCRATCH = [
    pltpu.VMEM((8, 2, CH), jnp.int32),
    pltpu.VMEM((CH, 32), jnp.float32),
    pltpu.VMEM((CH, 32), jnp.float32),
    pltpu.VMEM((CH, 32), jnp.float32),
    pltpu.VMEM((CH, 32), jnp.float32),
    pltpu.VMEM_SHARED((NP, 32), jnp.float32),
] + [pltpu.SemaphoreType.DMA] * 12


def _prop_feat_split(edges3d, zs_lo, zs_hi):
    cpt = NROWS // NS

    def body(edges_hbm, lo_hbm, hi_hbm, ulo_hbm, uhi_hbm,
             ib, rows0, rows1, rows2, rows3, acc, *sems):
        c = lax.axis_index("c")
        s = lax.axis_index("s")
        r0 = s * cpt
        base = s * STRIPE
        @pl.when(c == 0)
        def _():
            pltpu.sync_copy(lo_hbm.at[pl.ds(base, STRIPE)], acc.at[pl.ds(base, STRIPE)])

        @pl.when(c == 1)
        def _():
            pltpu.sync_copy(hi_hbm.at[pl.ds(base, STRIPE)], acc.at[pl.ds(base, STRIPE)])

        plsc.subcore_barrier()
        rows = (rows0, rows1, rows2, rows3)
        isems, gsems, ssems = sems[0:4], sems[4:8], sems[8:12]

        @pl.when(c == 0)
        def _():
            _scatter_loop(lo_hbm, edges_hbm, r0, ib, rows, isems, gsems,
                          ssems, acc, cpt)

        @pl.when(c == 1)
        def _():
            _scatter_loop(hi_hbm, edges_hbm, r0, ib, rows, isems, gsems,
                          ssems, acc, cpt)

        plsc.subcore_barrier()

        @pl.when(c == 0)
        def _():
            pltpu.sync_copy(acc.at[pl.ds(base, STRIPE)], ulo_hbm.at[pl.ds(base, STRIPE)])

        @pl.when(c == 1)
        def _():
            pltpu.sync_copy(acc.at[pl.ds(base, STRIPE)], uhi_hbm.at[pl.ds(base, STRIPE)])

    f = pl.kernel(
        body,
        out_type=[jax.ShapeDtypeStruct((NP, 32), jnp.float32),
                  jax.ShapeDtypeStruct((NP, 32), jnp.float32)],
        mesh=_mesh(),
        compiler_params=pltpu.CompilerParams(use_tc_tiling_on_sc=False),
        scratch_types=_PROP_SCRATCH,
    )
    return f(edges3d, zs_lo, zs_hi)


def _prop_edge_split(edges3d, zs):
    cpt = NROWS // (NC * NS)

    def body(edges_hbm, zs_hbm, u0_hbm, u1_hbm,
             ib, rows0, rows1, rows2, rows3, acc, *sems):
        c = lax.axis_index("c")
        s = lax.axis_index("s")
        r0 = (s * NC + c) * cpt
        base = s * STRIPE
        @pl.when(c == 0)
        def _():
            pltpu.sync_copy(zs_hbm.at[pl.ds(base, STRIPE)], acc.at[pl.ds(base, STRIPE)])

        @pl.when(c == 1)
        def _():
            _zero_rows(rows0, CH, 32)
            _zero_stripe(acc, base, rows0)

        plsc.subcore_barrier()
        _scatter_loop(zs_hbm, edges_hbm, r0, ib, (rows0, rows1, rows2, rows3),
                      sems[0:4], sems[4:8], sems[8:12], acc, cpt)
        plsc.subcore_barrier()

        @pl.when(c == 0)
        def _():
            pltpu.sync_copy(acc.at[pl.ds(base, STRIPE)], u0_hbm.at[pl.ds(base, STRIPE)])

        @pl.when(c == 1)
        def _():
            pltpu.sync_copy(acc.at[pl.ds(base, STRIPE)], u1_hbm.at[pl.ds(base, STRIPE)])

    f = pl.kernel(
        body,
        out_type=[jax.ShapeDtypeStruct((NP, 32), jnp.float32),
                  jax.ShapeDtypeStruct((NP, 32), jnp.float32)],
        mesh=_mesh(),
        compiler_params=pltpu.CompilerParams(use_tc_tiling_on_sc=False),
        scratch_types=_PROP_SCRATCH,
    )
    return f(edges3d, zs)


def _prop4_fused(edges3d, u3a, u3b, dv1d, b3):
    cpt = NROWS // (NC * NS)

    def body(edges_hbm, ua_hbm, ub_hbm, dv_hbm, b_hbm, z0_hbm, z1_hbm,
             t0_hbm, t1_hbm, ib, rows0, rows1, rows2, rows3, acc, bb, *sems):
        va, vb, dvb = rows1, rows2, rows3

        c = lax.axis_index("c")
        s = lax.axis_index("s")
        r0 = (s * NC + c) * cpt
        base = s * STRIPE
        pltpu.sync_copy(b_hbm, bb)
        b0 = bb[pl.ds(0, 16)]
        b1 = bb[pl.ds(16, 16)]
        nchk = STRIPE // CH
        rem = STRIPE - nchk * CH

        def fill(off, nrows):
            pltpu.sync_copy(ua_hbm.at[pl.ds(off, nrows)], va.at[pl.ds(0, nrows)])
            pltpu.sync_copy(ub_hbm.at[pl.ds(off, nrows)], vb.at[pl.ds(0, nrows)])
            pltpu.sync_copy(dv_hbm.at[pl.ds(off, nrows)], dvb.at[pl.ds(0, nrows)])

            def rowfn(r, _):
                d0 = dvb[r, pl.ds(0, 16)]
                d1 = dvb[r, pl.ds(16, 16)]
                x0 = jnp.maximum((va[r, pl.ds(0, 16)] + vb[r, pl.ds(0, 16)]) * d0 + b0, 0.0)
                x1 = jnp.maximum((va[r, pl.ds(16, 16)] + vb[r, pl.ds(16, 16)]) * d1 + b1, 0.0)
                va[r, pl.ds(0, 16)] = x0 * d0
                va[r, pl.ds(16, 16)] = x1 * d1
                return 0

            lax.fori_loop(0, nrows, rowfn, 0)

            @pl.when(c == 0)
            def _():
                pltpu.sync_copy(va.at[pl.ds(0, nrows)], z0_hbm.at[pl.ds(off, nrows)])
                pltpu.sync_copy(va.at[pl.ds(0, nrows)], acc.at[pl.ds(off, nrows)])

            @pl.when(c == 1)
            def _():
                pltpu.sync_copy(va.at[pl.ds(0, nrows)], z1_hbm.at[pl.ds(off, nrows)])

        for k in range(nchk):
            fill(base + k * CH, CH)
        fill(base + nchk * CH, rem)

        @pl.when(c == 1)
        def _():
            _zero_rows(rows0, CH, 32)
            _zero_stripe(acc, base, rows0)

        plsc.subcore_barrier()
        rows = (rows0, rows1, rows2, rows3)

        @pl.when(c == 0)
        def _():
            _scatter_loop(z0_hbm, edges_hbm, r0, ib, rows, sems[0:4],
                          sems[4:8], sems[8:12], acc, cpt)

        @pl.when(c == 1)
        def _():
            _scatter_loop(z1_hbm, edges_hbm, r0, ib, rows, sems[0:4],
                          sems[4:8], sems[8:12], acc, cpt)

        plsc.subcore_barrier()

        @pl.when(c == 0)
        def _():
            pltpu.sync_copy(acc.at[pl.ds(base, STRIPE)], t0_hbm.at[pl.ds(base, STRIPE)])

        @pl.when(c == 1)
        def _():
            pltpu.sync_copy(acc.at[pl.ds(base, STRIPE)], t1_hbm.at[pl.ds(base, STRIPE)])

    f = pl.kernel(
        body,
        out_type=[jax.ShapeDtypeStruct((NP, 32), jnp.float32)] * 4,
        mesh=_mesh(),
        compiler_params=pltpu.CompilerParams(use_tc_tiling_on_sc=False),
        scratch_types=_PROP_SCRATCH[:6] + [
            pltpu.VMEM((32,), jnp.float32),
        ] + [pltpu.SemaphoreType.DMA] * 12,
    )
    z0, z1, t0, t1 = f(edges3d, u3a, u3b, dv1d, b3)
    return t0, t1


def _row_spec(w):
    return pl.BlockSpec((R, w), lambda i: (i, 0))


def _full_spec(a, b):
    return pl.BlockSpec((a, b), lambda i: (0, 0))


def _tc1(x2, d0, d1, embed, W1):
    def body(x_ref, d0_ref, d1_ref, emb_ref, w1_ref, lo_ref, hi_ref, dv_ref,
             dvr_ref):
        deg = d0_ref[:, 0:1] + d1_ref[:, 0:1] + 1.0
        dv = lax.rsqrt(deg)
        iota = lax.broadcasted_iota(jnp.int32, (R, NUM_TYPES), 1)
        oh = (x_ref[...] == iota).astype(jnp.float32)
        ew = jnp.dot(emb_ref[...], w1_ref[...], preferred_element_type=jnp.float32)
        zs = dv * jnp.dot(oh, ew, preferred_element_type=jnp.float32)
        lo_ref[...] = zs[:, :32]
        hi_ref[...] = zs[:, 32:]
        dv_ref[...] = dv
        dvr_ref[...] = jnp.broadcast_to(dv, (R, 32))

    return pl.pallas_call(
        body,
        grid=(G,),
        in_specs=[_row_spec(1), _row_spec(16), _row_spec(16),
                  _full_spec(NUM_TYPES, H), _full_spec(H, H)],
        out_specs=[_row_spec(32), _row_spec(32), _row_spec(1), _row_spec(32)],
        out_shape=[jax.ShapeDtypeStruct((NP, 32), jnp.float32),
                   jax.ShapeDtypeStruct((NP, 32), jnp.float32),
                   jax.ShapeDtypeStruct((NP, 1), jnp.float32),
                   jax.ShapeDtypeStruct((NP, 32), jnp.float32)],
    )(x2, d0, d1, embed, W1)


def _tc_mid64(ulo, uhi, dv, W, b, out_w):
    def body(ulo_ref, uhi_ref, dv_ref, w_ref, b_ref, *outs):
        dvb = dv_ref[...]
        u = jnp.concatenate([ulo_ref[...], uhi_ref[...]], axis=1)
        h = jnp.maximum(dvb * u + b_ref[...], 0.0)
        zn = dvb * jnp.dot(h, w_ref[...], preferred_element_type=jnp.float32)
        if out_w == H:
            outs[0][...] = zn[:, :32]
            outs[1][...] = zn[:, 32:]
        else:
            outs[0][...] = zn

    if out_w == H:
        out_specs = [_row_spec(32), _row_spec(32)]
        out_shape = [jax.ShapeDtypeStruct((NP, 32), jnp.float32)] * 2
    else:
        out_specs = [_row_spec(32)]
        out_shape = [jax.ShapeDtypeStruct((NP, 32), jnp.float32)]
    return pl.pallas_call(
        body,
        grid=(G,),
        in_specs=[_row_spec(32)] * 2 + [_row_spec(1),
                  _full_spec(H, out_w), _full_spec(1, H)],
        out_specs=out_specs,
        out_shape=out_shape,
    )(ulo, uhi, dv, W, b)


def _tc5(u0, u1, dv, Wmu, bmu, Wls, bls):
    def body(u0_ref, u1_ref, dv_ref, wmu_ref, bmu_ref, wls_ref, bls_ref,
             mu_ref, ls_ref):
        t = dv_ref[...] * (u0_ref[...] + u1_ref[...])
        mu_ref[...] = jnp.dot(t, wmu_ref[...], preferred_element_type=jnp.float32) + bmu_ref[...]
        ls_ref[...] = jnp.dot(t, wls_ref[...], preferred_element_type=jnp.float32) + bls_ref[...]

    return pl.pallas_call(
        body,
        grid=(G,),
        in_specs=[_row_spec(32)] * 2 + [_row_spec(1),
                  _full_spec(OUT, OUT), _full_spec(1, OUT),
                  _full_spec(OUT, OUT), _full_spec(1, OUT)],
        out_specs=[_row_spec(32), _row_spec(32)],
        out_shape=[jax.ShapeDtypeStruct((NP, 32), jnp.float32)] * 2,
    )(u0, u1, dv, Wmu, bmu, Wls, bls)


def kernel(x, edge_index, embed, W1, b1, W2, b2, W3, b3, Wmu, bmu, Wls, bls):
    ei = edge_index.astype(jnp.int32)
    pad = jnp.full((2, EP - E), N, dtype=jnp.int32)
    ei = jnp.concatenate([ei, pad], axis=1)
    dst2d = ei[1].reshape(NROWS, CH)
    edges3d = jnp.stack([ei[0].reshape(NROWS, CH), ei[1].reshape(NROWS, CH)],
                        axis=1)
    x2 = jnp.pad(x.astype(jnp.int32), (0, NP - N)).reshape(NP, 1)

    d0, d1 = _deg_kernel(dst2d)
    zs1_lo, zs1_hi, dv, dv32 = _tc1(x2, d0, d1, embed, W1)
    u1_lo, u1_hi = _prop_feat_split(edges3d, zs1_lo, zs1_hi)
    zs2_lo, zs2_hi = _tc_mid64(u1_lo, u1_hi, dv, W2, b1.reshape(1, H), H)
    u2_lo, u2_hi = _prop_feat_split(edges3d, zs2_lo, zs2_hi)
    (zs3,) = _tc_mid64(u2_lo, u2_hi, dv, W3, b2.reshape(1, H), OUT)
    u3a, u3b = _prop_edge_split(edges3d, zs3)
    u4a, u4b = _prop4_fused(edges3d, u3a, u3b, dv32, b3)
    mu, ls = _tc5(u4a, u4b, dv, Wmu, bmu.reshape(1, OUT),
                  Wls, bls.reshape(1, OUT))
    return (mu[:N], ls[:N])
